# baseline (speedup 1.0000x reference)
import functools

import jax
import jax.numpy as jnp
import numpy as np
from jax import lax
from jax.experimental import pallas as pl
from jax.experimental.pallas import tpu as pltpu
from jax.experimental.pallas import tpu_sc as plsc

R = 512
F = 32
HW = R * R
N = HW
N4 = 4 * N

_k1d = np.exp(-0.5 * np.arange(-2, 3, dtype=np.float64) ** 2)
_k1d = (_k1d / _k1d.sum()).astype(np.float32)

_NB = 32


def _prep_body(coords_ref, featT_ref, idx_ref, upd_ref):
    c = coords_ref
    pa = jnp.clip((c[0] + 1.0) * 0.5, 0.0, 0.999) * (R - 1)
    pb = jnp.clip((c[1] + 1.0) * 0.5, 0.0, 0.999) * (R - 1)
    a0 = jnp.floor(pa)
    b0 = jnp.floor(pb)
    a0i = a0.astype(jnp.int32)
    b0i = b0.astype(jnp.int32)
    wa1 = pa - a0
    wa0 = (a0 + 1.0) - pa
    wb1 = pb - b0
    wb0 = (b0 + 1.0) - pb
    base = b0i * R + a0i
    idx_ref[...] = jnp.stack([base, base + R, base + 1, base + R + 1], axis=0)
    w4 = jnp.stack([wa0 * wb0, wa0 * wb1, wa1 * wb0, wa1 * wb1], axis=0)
    uf = featT_ref[...]
    upd = uf[:, None, :, :] * w4[None, :, :, :]
    upd_ref[...] = upd


def _prep(coords_p, featT_p):
    grid = (R // _NB,)
    return pl.pallas_call(
        _prep_body,
        grid=grid,
        in_specs=[
            pl.BlockSpec((2, _NB, R), lambda h: (0, h, 0)),
            pl.BlockSpec((F, _NB, R), lambda h: (0, h, 0)),
        ],
        out_specs=[
            pl.BlockSpec((4, _NB, R), lambda h: (0, h, 0)),
            pl.BlockSpec((F, 4, _NB, R), lambda h: (0, 0, h, 0)),
        ],
        out_shape=[
            jax.ShapeDtypeStruct((4, R, R), jnp.int32),
            jax.ShapeDtypeStruct((F, 4, R, R), jnp.float32),
        ],
    )(coords_p, featT_p)


_B = 4096
_TPB = N4 // 16
_SL = HW // 16


def _scatter_body(idx_hbm, upd_hbm, acc_hbm, sp0, sp1, sp2, sp3,
                  ia, ib, ua0, ua1, ua2, ua3, ub0, ub1, ub2, ub3,
                  ones_v, zeros_v, semA, semB, semS):
    sps = (sp0, sp1, sp2, sp3)
    ibufs = (ia, ib)
    ubufs = ((ua0, ua1, ua2, ua3), (ub0, ub1, ub2, ub3))
    sems = (semA, semB)
    cid = lax.axis_index("c")
    sid = lax.axis_index("s")
    nwin = _TPB // _B

    def _fill(i, carry):
        ones_v[pl.ds(i * 16, 16)] = jnp.full((16,), 1.0, jnp.float32)
        zeros_v[pl.ds(i * 16, 16)] = jnp.zeros((16,), jnp.float32)
        return carry

    lax.fori_loop(0, _B // 16, _fill, 0)

    off = sid * _SL
    for slot in range(4):
        for z in range(_SL // _B):
            pltpu.sync_copy(zeros_v, sps[slot].at[pl.ds(off + z * _B, _B)])
    plsc.subcore_barrier()

    def _pass(pc, carry):
        fbase = cid * 16 + pc * 4
        is_feat = pc < 4

        @pl.when(is_feat)
        def _feats():
            def _loads(wi, bsel):
                start = sid * _TPB + wi * _B
                hs = [pltpu.async_copy(idx_hbm.at[pl.ds(start, _B)],
                                       ibufs[bsel], sems[bsel])]
                for fl in range(4):
                    hs.append(pltpu.async_copy(
                        upd_hbm.at[fbase + fl, pl.ds(start, _B)],
                        ubufs[bsel][fl], sems[bsel]))
                return hs

            hs_cur = _loads(0, 0)
            for w in range(nwin):
                cb = w % 2
                hs_next = _loads(w + 1, 1 - cb) if w < nwin - 1 else None
                for h in hs_cur:
                    h.wait()
                ss = [pltpu.async_copy(ubufs[cb][fl], sps[fl].at[ibufs[cb]],
                                       semS, add=True) for fl in range(4)]
                for s in ss:
                    s.wait()
                hs_cur = hs_next

        @pl.when(jnp.logical_not(is_feat))
        def _counts():
            def _cload(wi, bsel):
                start = cid * (2 * N) + sid * (_TPB // 2) + wi * _B
                return pltpu.async_copy(idx_hbm.at[pl.ds(start, _B)],
                                        ibufs[bsel], sems[bsel])

            h_cur = _cload(0, 0)
            for w in range(nwin // 2):
                cb = w % 2
                h_next = _cload(w + 1, 1 - cb) if w < nwin // 2 - 1 else None
                h_cur.wait()
                pltpu.async_copy(ones_v, sp0.at[ibufs[cb]], semS,
                                 add=True).wait()
                h_cur = h_next

        plsc.subcore_barrier()

        @pl.when(is_feat)
        def _fdump():
            for fl in range(4):
                pltpu.sync_copy(sps[fl].at[pl.ds(off, _SL)],
                                acc_hbm.at[fbase + fl, pl.ds(off, _SL)])
                for z in range(_SL // _B):
                    pltpu.sync_copy(zeros_v,
                                    sps[fl].at[pl.ds(off + z * _B, _B)])

        @pl.when(jnp.logical_not(is_feat))
        def _cdump():
            pltpu.sync_copy(sp0.at[pl.ds(off, _SL)],
                            acc_hbm.at[32 + cid, pl.ds(off, _SL)])
            for z in range(_SL // _B):
                pltpu.sync_copy(zeros_v, sp0.at[pl.ds(off + z * _B, _B)])

        plsc.subcore_barrier()
        return carry

    lax.fori_loop(0, 5, _pass, 0)


def _scatter(idx, upd):
    mesh = plsc.VectorSubcoreMesh(core_axis_name="c", subcore_axis_name="s")
    k = pl.kernel(
        _scatter_body,
        out_type=jax.ShapeDtypeStruct((34, HW), jnp.float32),
        mesh=mesh,
        scratch_types=(
            [pltpu.VMEM_SHARED((HW,), jnp.float32) for _ in range(4)]
            + [pltpu.VMEM((_B,), jnp.int32) for _ in range(2)]
            + [pltpu.VMEM((_B,), jnp.float32) for _ in range(8)]
            + [pltpu.VMEM((_B,), jnp.float32),
               pltpu.VMEM((_B,), jnp.float32),
               pltpu.SemaphoreType.DMA,
               pltpu.SemaphoreType.DMA,
               pltpu.SemaphoreType.DMA]
        ),
    )
    return k(idx, upd)


_HB3 = 128


def _stats_body(a_ref, c0_ref, c1_ref, x_ref, sums_ref):
    f = pl.program_id(0)
    h = pl.program_id(1)
    cnt = c0_ref[0] + c1_ref[0] + 1e-6
    x = a_ref[0] / cnt
    x_ref[0] = x
    s1 = jnp.sum(x.reshape(-1, 128), axis=0)
    s2 = jnp.sum((x * x).reshape(-1, 128), axis=0)

    @pl.when(jnp.logical_and(f == 0, h == 0))
    def _init():
        sums_ref[...] = jnp.zeros_like(sums_ref)

    sums_ref[0, :] += s1
    sums_ref[1, :] += s2


def _stats(acc_p):
    grid = (F, R // _HB3)
    return pl.pallas_call(
        _stats_body,
        grid=grid,
        in_specs=[
            pl.BlockSpec((1, _HB3, R), lambda f, h: (f, h, 0)),
            pl.BlockSpec((1, _HB3, R), lambda f, h: (32, h, 0)),
            pl.BlockSpec((1, _HB3, R), lambda f, h: (33, h, 0)),
        ],
        out_specs=[
            pl.BlockSpec((1, _HB3, R), lambda f, h: (f, h, 0)),
            pl.BlockSpec((8, 128), lambda f, h: (0, 0)),
        ],
        out_shape=[
            jax.ShapeDtypeStruct((F, R, R), jnp.float32),
            jax.ShapeDtypeStruct((8, 128), jnp.float32),
        ],
    )(acc_p, acc_p, acc_p)


_HB4 = 128
_NH4 = R // _HB4


def _hshift(t, o):
    z = jnp.zeros((t.shape[0], abs(o)), jnp.float32)
    if o < 0:
        return jnp.concatenate([z, t[:, :R + o]], axis=1)
    if o > 0:
        return jnp.concatenate([t[:, o:], z], axis=1)
    return t


def _blur_body(xp_ref, xc_ref, xn_ref, wp_ref, wc_ref, wn_ref,
               bp_ref, bc_ref, bn_ref, pl_ref, scal_ref, out_ref):
    h = pl.program_id(1)
    mu = scal_ref[0, 0]
    inv = scal_ref[0, 1]

    def pad(pr, cr, nr):
        return jnp.concatenate([pr[0, _HB4 - 2:], cr[0], nr[0, :2]], axis=0)

    x = pad(xp_ref, xc_ref, xn_ref)
    w = pad(wp_ref, wc_ref, wn_ref)
    b = pad(bp_ref, bc_ref, bn_ref)
    z = ((x - mu) * inv) * w + b
    rid = h * _HB4 - 2 + lax.broadcasted_iota(jnp.int32, (_HB4 + 4, R), 0)
    z = jnp.where(jnp.logical_and(rid >= 0, rid < R), z, 0.0)
    t = jnp.zeros((_HB4, R), jnp.float32)
    for d in range(5):
        t = t + _k1d[d] * z[d:d + _HB4, :]
    o = jnp.zeros((_HB4, R), jnp.float32)
    for d in range(5):
        o = o + _k1d[d] * _hshift(t, d - 2)
    out_ref[0] = o + pl_ref[0]


def _blur(x_p, ln_w, ln_b, plane_p, scal):
    grid = (F, _NH4)
    hp = lambda f, h: (f, jnp.maximum(h - 1, 0), 0)
    hc = lambda f, h: (f, h, 0)
    hn = lambda f, h: (f, jnp.minimum(h + 1, _NH4 - 1), 0)
    blk = lambda m: pl.BlockSpec((1, _HB4, R), m)
    return pl.pallas_call(
        _blur_body,
        grid=grid,
        in_specs=[
            blk(hp), blk(hc), blk(hn),
            blk(hp), blk(hc), blk(hn),
            blk(hp), blk(hc), blk(hn),
            blk(hc),
            pl.BlockSpec((1, 2), lambda f, h: (0, 0), memory_space=pltpu.SMEM),
        ],
        out_specs=pl.BlockSpec((1, _HB4, R), hc),
        out_shape=jax.ShapeDtypeStruct((F, R, R), jnp.float32),
    )(x_p, x_p, x_p, ln_w, ln_w, ln_w, ln_b, ln_b, ln_b, plane_p, scal)


def kernel(gaussian_features, gaussian_xyz, plane_xy, plane_xz, plane_yz,
           ln_weight, ln_bias):
    gx = gaussian_xyz[:, 0].reshape(R, R)
    gy = gaussian_xyz[:, 1].reshape(R, R)
    gz = gaussian_xyz[:, 2].reshape(R, R)
    pairs = ((gx, gy), (gx, gz), (gy, gz))

    planes = (plane_xy, plane_xz, plane_yz)
    outs = []
    xs, scals = [], []
    for p in range(3):
        coords_p = jnp.stack(pairs[p])
        featT_p = gaussian_features[:, p, :].T.reshape(F, R, R)
        idx_p, upd_p = _prep(coords_p, featT_p)
        acc_p = _scatter(idx_p.reshape(N4), upd_p.reshape(F, N4))
        acc_p = acc_p.reshape(34, R, R)
        x_p, sums = _stats(acc_p)
        s1 = jnp.sum(sums[0])
        s2 = jnp.sum(sums[1])
        m = jnp.float32(F * HW)
        mu = s1 / m
        var = s2 / m - mu * mu
        inv = lax.rsqrt(var + 1e-5)
        xs.append(x_p)
        scals.append(jnp.stack([mu, inv]).reshape(1, 2))
    for p in range(3):
        o = _blur(xs[p], ln_weight, ln_bias, planes[p].reshape(F, R, R),
                  scals[p])
        outs.append(o)
    return jnp.stack(outs)[:, None]


# speedup vs baseline: 1.0087x; 1.0087x over previous
import functools

import jax
import jax.numpy as jnp
import numpy as np
from jax import lax
from jax.experimental import pallas as pl
from jax.experimental.pallas import tpu as pltpu
from jax.experimental.pallas import tpu_sc as plsc

R = 512
F = 32
HW = R * R
N = HW
N4 = 4 * N

_k1d = np.exp(-0.5 * np.arange(-2, 3, dtype=np.float64) ** 2)
_k1d = (_k1d / _k1d.sum()).astype(np.float32)

_NB = 32


def _prep_body(coords_ref, featT_ref, idx_ref, upd_ref):
    c = coords_ref[0]
    pa = jnp.clip((c[0] + 1.0) * 0.5, 0.0, 0.999) * (R - 1)
    pb = jnp.clip((c[1] + 1.0) * 0.5, 0.0, 0.999) * (R - 1)
    a0 = jnp.floor(pa)
    b0 = jnp.floor(pb)
    a0i = a0.astype(jnp.int32)
    b0i = b0.astype(jnp.int32)
    wa1 = pa - a0
    wa0 = (a0 + 1.0) - pa
    wb1 = pb - b0
    wb0 = (b0 + 1.0) - pb
    base = b0i * R + a0i
    idx_ref[0] = jnp.stack([base, base + R, base + 1, base + R + 1], axis=0)
    w4 = jnp.stack([wa0 * wb0, wa0 * wb1, wa1 * wb0, wa1 * wb1], axis=0)
    uf = featT_ref[0]
    upd = uf[:, None, :, :] * w4[None, :, :, :]
    upd_ref[0] = upd.reshape(4 * F, _NB, R)


def _prep(coords, featT):
    grid = (3, R // _NB)
    return pl.pallas_call(
        _prep_body,
        grid=grid,
        in_specs=[
            pl.BlockSpec((1, 2, _NB, R), lambda p, h: (p, 0, h, 0)),
            pl.BlockSpec((1, F, _NB, R), lambda p, h: (p, 0, h, 0)),
        ],
        out_specs=[
            pl.BlockSpec((1, 4, _NB, R), lambda p, h: (p, 0, h, 0)),
            pl.BlockSpec((1, 4 * F, _NB, R), lambda p, h: (p, 0, h, 0)),
        ],
        out_shape=[
            jax.ShapeDtypeStruct((3, 4, R, R), jnp.int32),
            jax.ShapeDtypeStruct((3, 4 * F, R, R), jnp.float32),
        ],
    )(coords, featT)


_B = 16384
_TPB = N4 // 16
_SL = HW // 16


def _scatter_body(idx_hbm, upd_hbm, acc_hbm, sp0, sp1, sp2, sp3,
                  idx_v, upd_v, ones_v, zeros_v):
    sps = (sp0, sp1, sp2, sp3)
    cid = lax.axis_index("c")
    sid = lax.axis_index("s")

    def _fill(i, carry):
        ones_v[pl.ds(i * 16, 16)] = jnp.full((16,), 1.0, jnp.float32)
        zeros_v[pl.ds(i * 16, 16)] = jnp.zeros((16,), jnp.float32)
        return carry

    lax.fori_loop(0, _B // 16, _fill, 0)

    off = sid * _SL
    for slot in range(4):
        pltpu.sync_copy(zeros_v, sps[slot].at[pl.ds(off, _SL)])
    plsc.subcore_barrier()

    def _pass(pc, carry):
        fbase = cid * 16 + pc * 4
        is_feat = pc < 4

        @pl.when(is_feat)
        def _feats():
            def _win(w, c2):
                start = sid * _TPB + w * _B
                pltpu.sync_copy(idx_hbm.at[pl.ds(start, _B)], idx_v)
                for fl in range(4):
                    pltpu.sync_copy(upd_hbm.at[fbase + fl, pl.ds(start, _B)],
                                    upd_v)
                    pltpu.sync_copy(upd_v, sps[fl].at[idx_v], add=True)
                return c2

            lax.fori_loop(0, _TPB // _B, _win, 0)

        @pl.when(jnp.logical_not(is_feat))
        def _counts():
            def _cwin(w, c2):
                start = cid * (2 * N) + sid * (_TPB // 2) + w * _B
                pltpu.sync_copy(idx_hbm.at[pl.ds(start, _B)], idx_v)
                pltpu.sync_copy(ones_v, sp0.at[idx_v], add=True)
                return c2

            lax.fori_loop(0, _TPB // 2 // _B, _cwin, 0)

        plsc.subcore_barrier()

        @pl.when(is_feat)
        def _fdump():
            for fl in range(4):
                pltpu.sync_copy(sps[fl].at[pl.ds(off, _SL)],
                                acc_hbm.at[fbase + fl, pl.ds(off, _SL)])
                pltpu.sync_copy(zeros_v, sps[fl].at[pl.ds(off, _SL)])

        @pl.when(jnp.logical_not(is_feat))
        def _cdump():
            pltpu.sync_copy(sp0.at[pl.ds(off, _SL)],
                            acc_hbm.at[32 + cid, pl.ds(off, _SL)])
            pltpu.sync_copy(zeros_v, sp0.at[pl.ds(off, _SL)])

        plsc.subcore_barrier()
        return carry

    lax.fori_loop(0, 5, _pass, 0)


def _scatter(idx, upd):
    mesh = plsc.VectorSubcoreMesh(core_axis_name="c", subcore_axis_name="s")
    k = pl.kernel(
        _scatter_body,
        out_type=jax.ShapeDtypeStruct((34, HW), jnp.float32),
        mesh=mesh,
        scratch_types=[
            pltpu.VMEM_SHARED((HW,), jnp.float32),
            pltpu.VMEM_SHARED((HW,), jnp.float32),
            pltpu.VMEM_SHARED((HW,), jnp.float32),
            pltpu.VMEM_SHARED((HW,), jnp.float32),
            pltpu.VMEM((_B,), jnp.int32),
            pltpu.VMEM((_B,), jnp.float32),
            pltpu.VMEM((_B,), jnp.float32),
            pltpu.VMEM((_SL,), jnp.float32),
        ],
    )
    return k(idx, upd)


_HB3 = 128


def _stats_body(a_ref, c0_ref, c1_ref, x_ref, sums_ref):
    f = pl.program_id(0)
    h = pl.program_id(1)
    cnt = c0_ref[0] + c1_ref[0] + 1e-6
    x = a_ref[0] / cnt
    x_ref[0] = x
    s1 = jnp.sum(x.reshape(-1, 128), axis=0)
    s2 = jnp.sum((x * x).reshape(-1, 128), axis=0)

    @pl.when(jnp.logical_and(f == 0, h == 0))
    def _init():
        sums_ref[...] = jnp.zeros_like(sums_ref)

    sums_ref[0, :] += s1
    sums_ref[1, :] += s2


def _stats(acc_p):
    grid = (F, R // _HB3)
    return pl.pallas_call(
        _stats_body,
        grid=grid,
        in_specs=[
            pl.BlockSpec((1, _HB3, R), lambda f, h: (f, h, 0)),
            pl.BlockSpec((1, _HB3, R), lambda f, h: (32, h, 0)),
            pl.BlockSpec((1, _HB3, R), lambda f, h: (33, h, 0)),
        ],
        out_specs=[
            pl.BlockSpec((1, _HB3, R), lambda f, h: (f, h, 0)),
            pl.BlockSpec((8, 128), lambda f, h: (0, 0)),
        ],
        out_shape=[
            jax.ShapeDtypeStruct((F, R, R), jnp.float32),
            jax.ShapeDtypeStruct((8, 128), jnp.float32),
        ],
    )(acc_p, acc_p, acc_p)


_HB4 = 128
_NH4 = R // _HB4


def _hshift(t, o):
    z = jnp.zeros((t.shape[0], abs(o)), jnp.float32)
    if o < 0:
        return jnp.concatenate([z, t[:, :R + o]], axis=1)
    if o > 0:
        return jnp.concatenate([t[:, o:], z], axis=1)
    return t


def _blur_body(xp_ref, xc_ref, xn_ref, wp_ref, wc_ref, wn_ref,
               bp_ref, bc_ref, bn_ref, pl_ref, scal_ref, out_ref):
    h = pl.program_id(1)
    mu = scal_ref[0, 0]
    inv = scal_ref[0, 1]

    def pad(pr, cr, nr):
        return jnp.concatenate([pr[0, _HB4 - 2:], cr[0], nr[0, :2]], axis=0)

    x = pad(xp_ref, xc_ref, xn_ref)
    w = pad(wp_ref, wc_ref, wn_ref)
    b = pad(bp_ref, bc_ref, bn_ref)
    z = ((x - mu) * inv) * w + b
    rid = h * _HB4 - 2 + lax.broadcasted_iota(jnp.int32, (_HB4 + 4, R), 0)
    z = jnp.where(jnp.logical_and(rid >= 0, rid < R), z, 0.0)
    t = jnp.zeros((_HB4, R), jnp.float32)
    for d in range(5):
        t = t + _k1d[d] * z[d:d + _HB4, :]
    o = jnp.zeros((_HB4, R), jnp.float32)
    for d in range(5):
        o = o + _k1d[d] * _hshift(t, d - 2)
    out_ref[0] = o + pl_ref[0]


def _blur(x_p, ln_w, ln_b, plane_p, scal):
    grid = (F, _NH4)
    hp = lambda f, h: (f, jnp.maximum(h - 1, 0), 0)
    hc = lambda f, h: (f, h, 0)
    hn = lambda f, h: (f, jnp.minimum(h + 1, _NH4 - 1), 0)
    blk = lambda m: pl.BlockSpec((1, _HB4, R), m)
    return pl.pallas_call(
        _blur_body,
        grid=grid,
        in_specs=[
            blk(hp), blk(hc), blk(hn),
            blk(hp), blk(hc), blk(hn),
            blk(hp), blk(hc), blk(hn),
            blk(hc),
            pl.BlockSpec((1, 2), lambda f, h: (0, 0), memory_space=pltpu.SMEM),
        ],
        out_specs=pl.BlockSpec((1, _HB4, R), hc),
        out_shape=jax.ShapeDtypeStruct((F, R, R), jnp.float32),
    )(x_p, x_p, x_p, ln_w, ln_w, ln_w, ln_b, ln_b, ln_b, plane_p, scal)


def kernel(gaussian_features, gaussian_xyz, plane_xy, plane_xz, plane_yz,
           ln_weight, ln_bias):
    gx = gaussian_xyz[:, 0].reshape(R, R)
    gy = gaussian_xyz[:, 1].reshape(R, R)
    gz = gaussian_xyz[:, 2].reshape(R, R)
    coords = jnp.stack([
        jnp.stack([gx, gy]), jnp.stack([gx, gz]), jnp.stack([gy, gz])])
    featT = jnp.transpose(gaussian_features, (1, 2, 0)).reshape(3, F, R, R)

    idx, upd = _prep(coords, featT)
    idx = idx.reshape(3, N4)
    upd = upd.reshape(3, F, N4)

    planes = (plane_xy, plane_xz, plane_yz)
    outs = []
    xs, scals = [], []
    for p in range(3):
        acc_p = _scatter(idx[p], upd[p]).reshape(34, R, R)
        x_p, sums = _stats(acc_p)
        s1 = jnp.sum(sums[0])
        s2 = jnp.sum(sums[1])
        m = jnp.float32(F * HW)
        mu = s1 / m
        var = s2 / m - mu * mu
        inv = lax.rsqrt(var + 1e-5)
        xs.append(x_p)
        scals.append(jnp.stack([mu, inv]).reshape(1, 2))
    for p in range(3):
        o = _blur(xs[p], ln_weight, ln_bias, planes[p].reshape(F, R, R),
                  scals[p])
        outs.append(o)
    return jnp.stack(outs)[:, None]
